# baseline (speedup 1.0000x reference)
import functools

import jax
import jax.numpy as jnp
from jax import lax
from jax.experimental import pallas as pl
from jax.experimental.pallas import tpu as pltpu
from jax.experimental.pallas import tpu_sc as plsc

N = 10000
D = 128
NC = 2
NS = 16
NW = NC * NS
CH = 128
SLAB = 40
EPT = 20480
E_PAD = NS * EPT
IDXROWS = E_PAD // CH
ACC_ROWS = 10112
RPT = ACC_ROWS // NS
RB = 2048

_f32 = jnp.float32

_SLABS = [(o, min(CH, RPT - o)) for o in range(0, RPT, CH)]


def _zero_rbuf(rb):
    @pl.loop(0, CH * (D // 16))
    def _z(i):
        rb[i // (D // 16), pl.ds((i % (D // 16)) * 16, 16)] = (
            jnp.zeros((16,), _f32))


def _agg(src_h, dst_h, tbl_h, s_out, c_out, acc, cnt, sidx, didx, rbufs,
         onesb, cbuf, gsems, ssems, sid, row0, nstages, with_counts):
    _zero_rbuf(rbufs[0])
    for off, n in _SLABS:
        pltpu.sync_copy(rbufs[0].at[pl.ds(0, n), :],
                        acc.at[pl.ds(sid * RPT + off, n), :])
    if with_counts:
        for off, n in _SLABS:
            pltpu.sync_copy(rbufs[0].at[0, pl.ds(0, n)],
                            cnt.at[pl.ds(sid * RPT + off, n)])
    plsc.subcore_barrier()

    for st in range(nstages):
        r0 = row0 + st * SLAB
        pltpu.sync_copy(src_h.at[pl.ds(r0, SLAB), :], sidx)
        pltpu.sync_copy(dst_h.at[pl.ds(r0, SLAB), :], didx)
        pltpu.async_copy(tbl_h.at[sidx.at[0]], rbufs[0], gsems[0])

        @pl.loop(0, SLAB // 2)
        def _pair(p):
            for b in range(2):
                j = 2 * p + b
                o = 1 - b
                pltpu.make_async_copy(
                    tbl_h.at[sidx.at[b]], rbufs[b], gsems[b]).wait()

                @pl.when(j > 0)
                def _wprev():
                    pltpu.make_async_copy(
                        rbufs[o], acc.at[didx.at[0]], ssems[o]).wait()

                @pl.when(j + 1 < SLAB)
                def _pf():
                    pltpu.async_copy(tbl_h.at[sidx.at[j + 1]], rbufs[o],
                                     gsems[o])

                pltpu.async_copy(rbufs[b], acc.at[didx.at[j]], ssems[b],
                                 add=True)
                if with_counts:
                    pltpu.sync_copy(onesb, cnt.at[didx.at[j]], add=True)

        pltpu.make_async_copy(rbufs[1], acc.at[didx.at[0]], ssems[1]).wait()

    plsc.subcore_barrier()
    for off, n in _SLABS:
        pltpu.sync_copy(acc.at[pl.ds(sid * RPT + off, n), :],
                        rbufs[0].at[pl.ds(0, n), :])
        pltpu.sync_copy(rbufs[0].at[pl.ds(0, n), :],
                        s_out.at[pl.ds(sid * RPT + off, n), :])
    if with_counts:
        pltpu.sync_copy(cnt.at[pl.ds(sid * RPT, RPT)], cbuf)
        pltpu.sync_copy(cbuf, c_out.at[pl.ds(sid * RPT, RPT)])
    plsc.subcore_barrier()


def _init_ones(onesb):
    @pl.loop(0, CH // 16)
    def _o(i):
        onesb[pl.ds(i * 16, 16)] = jnp.ones((16,), _f32)


def _mesh():
    return plsc.VectorSubcoreMesh(core_axis_name="c", subcore_axis_name="s")


def _sc_two_aggs(si, di, su, du, x):
    @functools.partial(
        pl.kernel,
        out_type=(
            jax.ShapeDtypeStruct((ACC_ROWS, D), _f32),
            jax.ShapeDtypeStruct((ACC_ROWS,), _f32),
            jax.ShapeDtypeStruct((ACC_ROWS, D), _f32),
            jax.ShapeDtypeStruct((ACC_ROWS,), _f32),
        ),
        mesh=_mesh(),
        scratch_types=[
            pltpu.VMEM_SHARED((ACC_ROWS, D), _f32),
            pltpu.VMEM_SHARED((ACC_ROWS,), _f32),
            pltpu.VMEM((SLAB, CH), jnp.int32),
            pltpu.VMEM((SLAB, CH), jnp.int32),
            pltpu.VMEM((CH, D), _f32),
            pltpu.VMEM((CH, D), _f32),
            pltpu.VMEM((CH,), _f32),
            pltpu.VMEM((RPT,), _f32),
            pltpu.SemaphoreType.DMA,
            pltpu.SemaphoreType.DMA,
            pltpu.SemaphoreType.DMA,
            pltpu.SemaphoreType.DMA,
        ],
    )
    def body(si_h, di_h, su_h, du_h, x_h, s1_h, c1_h, s2_h, c2_h,
             acc, cnt, sidx, didx, r0, r1, onesb, cbuf, g0, g1, q0, q1):
        cid = lax.axis_index("c")
        sid = lax.axis_index("s")
        rbufs = (r0, r1)
        gsems = (g0, g1)
        ssems = (q0, q1)
        _init_ones(onesb)
        row0 = sid * (EPT // CH)

        @pl.when(cid == 0)
        def _ii():
            _agg(si_h, di_h, x_h, s1_h, c1_h, acc, cnt, sidx, didx, rbufs,
                 onesb, cbuf, gsems, ssems, sid, row0, EPT // CH // SLAB,
                 True)

        @pl.when(cid == 1)
        def _iu():
            _agg(su_h, du_h, x_h, s2_h, c2_h, acc, cnt, sidx, didx, rbufs,
                 onesb, cbuf, gsems, ssems, sid, row0, EPT // CH // SLAB,
                 True)

    return body(si, di, su, du, x)


def _sc_one_agg(su, du, x):
    @functools.partial(
        pl.kernel,
        out_type=jax.ShapeDtypeStruct((NC, ACC_ROWS, D), _f32),
        mesh=_mesh(),
        scratch_types=[
            pltpu.VMEM_SHARED((ACC_ROWS, D), _f32),
            pltpu.VMEM((SLAB, CH), jnp.int32),
            pltpu.VMEM((SLAB, CH), jnp.int32),
            pltpu.VMEM((CH, D), _f32),
            pltpu.VMEM((CH, D), _f32),
            pltpu.SemaphoreType.DMA,
            pltpu.SemaphoreType.DMA,
            pltpu.SemaphoreType.DMA,
            pltpu.SemaphoreType.DMA,
        ],
    )
    def body(su_h, du_h, x_h, s3_h, acc, sidx, didx, r0, r1, g0, g1, q0, q1):
        cid = lax.axis_index("c")
        sid = lax.axis_index("s")
        wid = cid * NS + sid
        row0 = wid * (EPT // NC // CH)
        _agg(su_h, du_h, x_h, s3_h.at[cid], None, acc, None, sidx, didx,
             (r0, r1), None, None, (g0, g1), (q0, q1), sid, row0,
             EPT // NC // CH // SLAB, False)

    return body(su, du, x)


def _row_spec():
    return pl.BlockSpec((RB, D), lambda i: (i, 0))


def _full_spec(shape):
    nd = len(shape)
    return pl.BlockSpec(shape, lambda i: (0,) * nd)


def _cnt_spec():
    return pl.BlockSpec((1, RB), lambda i: (0, i))


_GRID = ((N + RB - 1) // RB,)


def _tc_roots(xi, xu, w1r, w2r, b1, b2):
    def body(xi_r, xu_r, w1r_r, w2r_r, b1_r, b2_r, r1_r, r2_r):
        r1_r[...] = jnp.dot(xi_r[...], w1r_r[...],
                            preferred_element_type=_f32) + b1_r[...]
        r2_r[...] = jnp.dot(xu_r[...], w2r_r[...],
                            preferred_element_type=_f32) + b2_r[...]

    o = jax.ShapeDtypeStruct((N, D), _f32)
    return pl.pallas_call(
        body,
        grid=_GRID,
        in_specs=[_row_spec(), _row_spec(), _full_spec((D, D)),
                  _full_spec((D, D)), _full_spec((1, D)), _full_spec((1, D))],
        out_specs=[_row_spec()] * 2,
        out_shape=[o, o],
    )(xi, xu, w1r, w2r, b1, b2)


def _tc_mid(t1, c1, r1, w1l, t2, c2, r2, w2l, w3r, b3):
    def body(t1_r, c1_r, r1_r, w1l_r, t2_r, c2_r, r2_r, w2l_r, w3r_r, b3_r,
             ix_r, r3_r):
        m1 = t1_r[...] / jnp.maximum(c1_r[...][0], 1.0)[:, None]
        ix_r[...] = jnp.maximum(
            jnp.dot(m1, w1l_r[...], preferred_element_type=_f32) + r1_r[...],
            0.0)
        m2 = t2_r[...] / jnp.maximum(c2_r[...][0], 1.0)[:, None]
        u2 = jnp.maximum(
            jnp.dot(m2, w2l_r[...], preferred_element_type=_f32) + r2_r[...],
            0.0)
        r3_r[...] = jnp.dot(u2, w3r_r[...],
                            preferred_element_type=_f32) + b3_r[...]

    o = jax.ShapeDtypeStruct((N, D), _f32)
    return pl.pallas_call(
        body,
        grid=_GRID,
        in_specs=[_row_spec(), _cnt_spec(), _row_spec(), _full_spec((D, D)),
                  _row_spec(), _cnt_spec(), _row_spec(), _full_spec((D, D)),
                  _full_spec((D, D)), _full_spec((1, D))],
        out_specs=[_row_spec()] * 2,
        out_shape=[o, o],
    )(t1, c1, r1, w1l, t2, c2, r2, w2l, w3r, b3)


def _tc_final(t3p, c2, r3, w3l, wlin, blin):
    def body(t3_r, c2_r, r3_r, w3l_r, wl_r, bl_r, out_r):
        t3 = t3_r[...][0] + t3_r[...][1]
        m3 = t3 / jnp.maximum(c2_r[...][0], 1.0)[:, None]
        u3 = jnp.maximum(
            jnp.dot(m3, w3l_r[...], preferred_element_type=_f32) + r3_r[...],
            0.0)
        out_r[...] = jnp.dot(u3, wl_r[...],
                             preferred_element_type=_f32) + bl_r[...]

    return pl.pallas_call(
        body,
        grid=_GRID,
        in_specs=[pl.BlockSpec((NC, RB, D), lambda i: (0, i, 0)), _cnt_spec(),
                  _row_spec(), _full_spec((D, D)), _full_spec((D, D)),
                  _full_spec((1, D))],
        out_specs=_row_spec(),
        out_shape=jax.ShapeDtypeStruct((N, D), _f32),
    )(t3p, c2, r3, w3l, wlin, blin)


def _prep_edges(edge_index):
    src = edge_index[0].astype(jnp.int32)
    dst = edge_index[1].astype(jnp.int32)
    npad = E_PAD - src.shape[0]
    ar = jnp.arange(npad, dtype=jnp.int32)
    pad_src = ar % N
    pad_dst = N + ar % (ACC_ROWS - N)
    src = jnp.concatenate([src, pad_src]).reshape(IDXROWS, CH)
    dst = jnp.concatenate([dst, pad_dst]).reshape(IDXROWS, CH)
    return src, dst


def kernel(x_item, x_user, edge_index_ii, edge_index_iu, W1l, b1l, W1r,
           W2l, b2l, W2r, W3l, b3l, W3r, Wlin, blin):
    si, di = _prep_edges(edge_index_ii)
    su, du = _prep_edges(edge_index_iu)
    b1 = b1l.reshape(1, D)
    b2 = b2l.reshape(1, D)
    b3 = b3l.reshape(1, D)
    bl = blin.reshape(1, D)

    t1, c1, t2, c2 = _sc_two_aggs(si, di, su, du, x_item)
    c1 = c1.reshape(1, ACC_ROWS)
    c2 = c2.reshape(1, ACC_ROWS)
    r1, r2 = _tc_roots(x_item, x_user, W1r.T, W2r.T, b1, b2)
    item_x, r3 = _tc_mid(t1, c1, r1, W1l.T, t2, c2, r2, W2l.T, W3r.T, b3)
    t3p = _sc_one_agg(su, du, item_x)
    return _tc_final(t3p, c2, r3, W3l.T, Wlin.T, bl)


# speedup vs baseline: 1.1579x; 1.1579x over previous
import functools

import jax
import jax.numpy as jnp
from jax import lax
from jax.experimental import pallas as pl
from jax.experimental.pallas import tpu as pltpu
from jax.experimental.pallas import tpu_sc as plsc

N = 10000
D = 128
NC = 2
NS = 16
NW = NC * NS
CH = 128
SLAB = 40
EPT = 20480
E_PAD = NS * EPT
IDXROWS = E_PAD // CH
ACC_ROWS = 10112
RPT = ACC_ROWS // NS
RB = 2048

_f32 = jnp.float32

_SLABS = [(o, min(CH, RPT - o)) for o in range(0, RPT, CH)]


def _zero_rbuf(rb):
    @pl.loop(0, CH * (D // 16))
    def _z(i):
        rb[i // (D // 16), pl.ds((i % (D // 16)) * 16, 16)] = (
            jnp.zeros((16,), _f32))


def _agg(src_h, dst_h, tbl_h, s_out, c_out, acc, cnt, sidx, didx, rbufs,
         onesb, cbuf, gsems, ssems, sid, row0, nstages, with_counts):
    _zero_rbuf(rbufs[0])
    for off, n in _SLABS:
        pltpu.sync_copy(rbufs[0].at[pl.ds(0, n), :],
                        acc.at[pl.ds(sid * RPT + off, n), :])
    if with_counts:
        for off, n in _SLABS:
            pltpu.sync_copy(rbufs[0].at[0, pl.ds(0, n)],
                            cnt.at[pl.ds(sid * RPT + off, n)])
    plsc.subcore_barrier()

    for st in range(nstages):
        r0 = row0 + st * SLAB
        pltpu.sync_copy(src_h.at[pl.ds(r0, SLAB), :], sidx)
        pltpu.sync_copy(dst_h.at[pl.ds(r0, SLAB), :], didx)
        for b in range(2):
            pltpu.async_copy(tbl_h.at[sidx.at[b]], rbufs[b], gsems[b])

        @pl.loop(0, SLAB // 2)
        def _pair(p):
            for b in range(2):
                j = 2 * p + b
                pltpu.make_async_copy(
                    tbl_h.at[sidx.at[b]], rbufs[b], gsems[b]).wait()
                pltpu.sync_copy(rbufs[b], acc.at[didx.at[j]], add=True)
                if with_counts:
                    pltpu.sync_copy(onesb, cnt.at[didx.at[j]], add=True)

                @pl.when(j + 2 < SLAB)
                def _pf():
                    pltpu.async_copy(tbl_h.at[sidx.at[j + 2]], rbufs[b],
                                     gsems[b])

    plsc.subcore_barrier()
    for off, n in _SLABS:
        pltpu.sync_copy(acc.at[pl.ds(sid * RPT + off, n), :],
                        rbufs[0].at[pl.ds(0, n), :])
        pltpu.sync_copy(rbufs[0].at[pl.ds(0, n), :],
                        s_out.at[pl.ds(sid * RPT + off, n), :])
    if with_counts:
        pltpu.sync_copy(cnt.at[pl.ds(sid * RPT, RPT)], cbuf)
        pltpu.sync_copy(cbuf, c_out.at[pl.ds(sid * RPT, RPT)])
    plsc.subcore_barrier()


def _init_ones(onesb):
    @pl.loop(0, CH // 16)
    def _o(i):
        onesb[pl.ds(i * 16, 16)] = jnp.ones((16,), _f32)


def _mesh():
    return plsc.VectorSubcoreMesh(core_axis_name="c", subcore_axis_name="s")


def _sc_two_aggs(si, di, su, du, x):
    @functools.partial(
        pl.kernel,
        out_type=(
            jax.ShapeDtypeStruct((ACC_ROWS, D), _f32),
            jax.ShapeDtypeStruct((ACC_ROWS,), _f32),
            jax.ShapeDtypeStruct((ACC_ROWS, D), _f32),
            jax.ShapeDtypeStruct((ACC_ROWS,), _f32),
        ),
        mesh=_mesh(),
        scratch_types=[
            pltpu.VMEM_SHARED((ACC_ROWS, D), _f32),
            pltpu.VMEM_SHARED((ACC_ROWS,), _f32),
            pltpu.VMEM((SLAB, CH), jnp.int32),
            pltpu.VMEM((SLAB, CH), jnp.int32),
            pltpu.VMEM((CH, D), _f32),
            pltpu.VMEM((CH, D), _f32),
            pltpu.VMEM((CH,), _f32),
            pltpu.VMEM((RPT,), _f32),
            pltpu.SemaphoreType.DMA,
            pltpu.SemaphoreType.DMA,
            pltpu.SemaphoreType.DMA,
            pltpu.SemaphoreType.DMA,
        ],
    )
    def body(si_h, di_h, su_h, du_h, x_h, s1_h, c1_h, s2_h, c2_h,
             acc, cnt, sidx, didx, r0, r1, onesb, cbuf, g0, g1, q0, q1):
        cid = lax.axis_index("c")
        sid = lax.axis_index("s")
        rbufs = (r0, r1)
        gsems = (g0, g1)
        ssems = (q0, q1)
        _init_ones(onesb)
        row0 = sid * (EPT // CH)

        @pl.when(cid == 0)
        def _ii():
            _agg(si_h, di_h, x_h, s1_h, c1_h, acc, cnt, sidx, didx, rbufs,
                 onesb, cbuf, gsems, ssems, sid, row0, EPT // CH // SLAB,
                 True)

        @pl.when(cid == 1)
        def _iu():
            _agg(su_h, du_h, x_h, s2_h, c2_h, acc, cnt, sidx, didx, rbufs,
                 onesb, cbuf, gsems, ssems, sid, row0, EPT // CH // SLAB,
                 True)

    return body(si, di, su, du, x)


def _sc_one_agg(su, du, x):
    @functools.partial(
        pl.kernel,
        out_type=jax.ShapeDtypeStruct((NC, ACC_ROWS, D), _f32),
        mesh=_mesh(),
        scratch_types=[
            pltpu.VMEM_SHARED((ACC_ROWS, D), _f32),
            pltpu.VMEM((SLAB, CH), jnp.int32),
            pltpu.VMEM((SLAB, CH), jnp.int32),
            pltpu.VMEM((CH, D), _f32),
            pltpu.VMEM((CH, D), _f32),
            pltpu.SemaphoreType.DMA,
            pltpu.SemaphoreType.DMA,
            pltpu.SemaphoreType.DMA,
            pltpu.SemaphoreType.DMA,
        ],
    )
    def body(su_h, du_h, x_h, s3_h, acc, sidx, didx, r0, r1, g0, g1, q0, q1):
        cid = lax.axis_index("c")
        sid = lax.axis_index("s")
        wid = cid * NS + sid
        row0 = wid * (EPT // NC // CH)
        _agg(su_h, du_h, x_h, s3_h.at[cid], None, acc, None, sidx, didx,
             (r0, r1), None, None, (g0, g1), (q0, q1), sid, row0,
             EPT // NC // CH // SLAB, False)

    return body(su, du, x)


def _row_spec():
    return pl.BlockSpec((RB, D), lambda i: (i, 0))


def _full_spec(shape):
    nd = len(shape)
    return pl.BlockSpec(shape, lambda i: (0,) * nd)


def _cnt_spec():
    return pl.BlockSpec((1, RB), lambda i: (0, i))


_GRID = ((N + RB - 1) // RB,)


def _tc_roots(xi, xu, w1r, w2r, b1, b2):
    def body(xi_r, xu_r, w1r_r, w2r_r, b1_r, b2_r, r1_r, r2_r):
        r1_r[...] = jnp.dot(xi_r[...], w1r_r[...],
                            preferred_element_type=_f32) + b1_r[...]
        r2_r[...] = jnp.dot(xu_r[...], w2r_r[...],
                            preferred_element_type=_f32) + b2_r[...]

    o = jax.ShapeDtypeStruct((N, D), _f32)
    return pl.pallas_call(
        body,
        grid=_GRID,
        in_specs=[_row_spec(), _row_spec(), _full_spec((D, D)),
                  _full_spec((D, D)), _full_spec((1, D)), _full_spec((1, D))],
        out_specs=[_row_spec()] * 2,
        out_shape=[o, o],
    )(xi, xu, w1r, w2r, b1, b2)


def _tc_mid(t1, c1, r1, w1l, t2, c2, r2, w2l, w3r, b3):
    def body(t1_r, c1_r, r1_r, w1l_r, t2_r, c2_r, r2_r, w2l_r, w3r_r, b3_r,
             ix_r, r3_r):
        m1 = t1_r[...] / jnp.maximum(c1_r[...][0], 1.0)[:, None]
        ix_r[...] = jnp.maximum(
            jnp.dot(m1, w1l_r[...], preferred_element_type=_f32) + r1_r[...],
            0.0)
        m2 = t2_r[...] / jnp.maximum(c2_r[...][0], 1.0)[:, None]
        u2 = jnp.maximum(
            jnp.dot(m2, w2l_r[...], preferred_element_type=_f32) + r2_r[...],
            0.0)
        r3_r[...] = jnp.dot(u2, w3r_r[...],
                            preferred_element_type=_f32) + b3_r[...]

    o = jax.ShapeDtypeStruct((N, D), _f32)
    return pl.pallas_call(
        body,
        grid=_GRID,
        in_specs=[_row_spec(), _cnt_spec(), _row_spec(), _full_spec((D, D)),
                  _row_spec(), _cnt_spec(), _row_spec(), _full_spec((D, D)),
                  _full_spec((D, D)), _full_spec((1, D))],
        out_specs=[_row_spec()] * 2,
        out_shape=[o, o],
    )(t1, c1, r1, w1l, t2, c2, r2, w2l, w3r, b3)


def _tc_final(t3p, c2, r3, w3l, wlin, blin):
    def body(t3_r, c2_r, r3_r, w3l_r, wl_r, bl_r, out_r):
        t3 = t3_r[...][0] + t3_r[...][1]
        m3 = t3 / jnp.maximum(c2_r[...][0], 1.0)[:, None]
        u3 = jnp.maximum(
            jnp.dot(m3, w3l_r[...], preferred_element_type=_f32) + r3_r[...],
            0.0)
        out_r[...] = jnp.dot(u3, wl_r[...],
                             preferred_element_type=_f32) + bl_r[...]

    return pl.pallas_call(
        body,
        grid=_GRID,
        in_specs=[pl.BlockSpec((NC, RB, D), lambda i: (0, i, 0)), _cnt_spec(),
                  _row_spec(), _full_spec((D, D)), _full_spec((D, D)),
                  _full_spec((1, D))],
        out_specs=_row_spec(),
        out_shape=jax.ShapeDtypeStruct((N, D), _f32),
    )(t3p, c2, r3, w3l, wlin, blin)


def _prep_edges(edge_index):
    src = edge_index[0].astype(jnp.int32)
    dst = edge_index[1].astype(jnp.int32)
    npad = E_PAD - src.shape[0]
    ar = jnp.arange(npad, dtype=jnp.int32)
    pad_src = ar % N
    pad_dst = N + ar % (ACC_ROWS - N)
    src = jnp.concatenate([src, pad_src]).reshape(IDXROWS, CH)
    dst = jnp.concatenate([dst, pad_dst]).reshape(IDXROWS, CH)
    return src, dst


def kernel(x_item, x_user, edge_index_ii, edge_index_iu, W1l, b1l, W1r,
           W2l, b2l, W2r, W3l, b3l, W3r, Wlin, blin):
    si, di = _prep_edges(edge_index_ii)
    su, du = _prep_edges(edge_index_iu)
    b1 = b1l.reshape(1, D)
    b2 = b2l.reshape(1, D)
    b3 = b3l.reshape(1, D)
    bl = blin.reshape(1, D)

    t1, c1, t2, c2 = _sc_two_aggs(si, di, su, du, x_item)
    c1 = c1.reshape(1, ACC_ROWS)
    c2 = c2.reshape(1, ACC_ROWS)
    r1, r2 = _tc_roots(x_item, x_user, W1r.T, W2r.T, b1, b2)
    item_x, r3 = _tc_mid(t1, c1, r1, W1l.T, t2, c2, r2, W2l.T, W3r.T, b3)
    t3p = _sc_one_agg(su, du, item_x)
    return _tc_final(t3p, c2, r3, W3l.T, Wlin.T, bl)


# speedup vs baseline: 1.1603x; 1.0021x over previous
import functools

import jax
import jax.numpy as jnp
from jax import lax
from jax.experimental import pallas as pl
from jax.experimental.pallas import tpu as pltpu
from jax.experimental.pallas import tpu_sc as plsc

N = 10000
D = 128
NC = 2
NS = 16
NW = NC * NS
CH = 128
SLAB = 40
EPT = 20480
E_PAD = NS * EPT
IDXROWS = E_PAD // CH
ACC_ROWS = 10112
RPT = ACC_ROWS // NS
RB = 2048

_f32 = jnp.float32

_SLABS = [(o, min(CH, RPT - o)) for o in range(0, RPT, CH)]


def _zero_rbuf(rb):
    @pl.loop(0, CH * (D // 16))
    def _z(i):
        rb[i // (D // 16), pl.ds((i % (D // 16)) * 16, 16)] = (
            jnp.zeros((16,), _f32))


def _agg(src_h, dst_h, tbl_h, s_out, c_out, acc, cnt, sidx, didx, rbufs,
         onesb, cbuf, gsems, ssems, sid, row0, nstages, with_counts):
    _zero_rbuf(rbufs[0])
    for off, n in _SLABS:
        pltpu.sync_copy(rbufs[0].at[pl.ds(0, n), :],
                        acc.at[pl.ds(sid * RPT + off, n), :])
    if with_counts:
        for off, n in _SLABS:
            pltpu.sync_copy(rbufs[0].at[0, pl.ds(0, n)],
                            cnt.at[pl.ds(sid * RPT + off, n)])
    plsc.subcore_barrier()

    for st in range(nstages):
        r0 = row0 + st * SLAB
        pltpu.sync_copy(src_h.at[pl.ds(r0, SLAB), :], sidx)
        pltpu.sync_copy(dst_h.at[pl.ds(r0, SLAB), :], didx)
        for b in range(2):
            pltpu.async_copy(tbl_h.at[sidx.at[b]], rbufs[b], gsems[b])

        @pl.loop(0, SLAB // 2)
        def _pair(p):
            for b in range(2):
                j = 2 * p + b
                pltpu.make_async_copy(
                    tbl_h.at[sidx.at[b]], rbufs[b], gsems[b]).wait()
                pltpu.sync_copy(rbufs[b], acc.at[didx.at[j]], add=True)
                if with_counts:
                    pltpu.sync_copy(onesb, cnt.at[didx.at[j]], add=True)

                @pl.when(j + 2 < SLAB)
                def _pf():
                    pltpu.async_copy(tbl_h.at[sidx.at[j + 2]], rbufs[b],
                                     gsems[b])

    plsc.subcore_barrier()
    for off, n in _SLABS:
        pltpu.sync_copy(acc.at[pl.ds(sid * RPT + off, n), :],
                        rbufs[0].at[pl.ds(0, n), :])
        pltpu.sync_copy(rbufs[0].at[pl.ds(0, n), :],
                        s_out.at[pl.ds(sid * RPT + off, n), :])
    if with_counts:
        pltpu.sync_copy(cnt.at[pl.ds(sid * RPT, RPT)], cbuf)
        pltpu.sync_copy(cbuf, c_out.at[pl.ds(sid * RPT, RPT)])
    plsc.subcore_barrier()


def _init_ones(onesb):
    @pl.loop(0, CH // 16)
    def _o(i):
        onesb[pl.ds(i * 16, 16)] = jnp.ones((16,), _f32)


def _mesh():
    return plsc.VectorSubcoreMesh(core_axis_name="c", subcore_axis_name="s")


def _sc_two_aggs(si, di, su, du, x):
    @functools.partial(
        pl.kernel,
        out_type=(
            jax.ShapeDtypeStruct((ACC_ROWS, D), _f32),
            jax.ShapeDtypeStruct((ACC_ROWS,), _f32),
            jax.ShapeDtypeStruct((ACC_ROWS, D), _f32),
            jax.ShapeDtypeStruct((ACC_ROWS,), _f32),
        ),
        mesh=_mesh(),
        scratch_types=[
            pltpu.VMEM_SHARED((ACC_ROWS, D), _f32),
            pltpu.VMEM_SHARED((ACC_ROWS,), _f32),
            pltpu.VMEM((SLAB, CH), jnp.int32),
            pltpu.VMEM((SLAB, CH), jnp.int32),
            pltpu.VMEM((CH, D), _f32),
            pltpu.VMEM((CH, D), _f32),
            pltpu.VMEM((CH,), _f32),
            pltpu.VMEM((RPT,), _f32),
            pltpu.SemaphoreType.DMA,
            pltpu.SemaphoreType.DMA,
            pltpu.SemaphoreType.DMA,
            pltpu.SemaphoreType.DMA,
        ],
    )
    def body(si_h, di_h, su_h, du_h, x_h, s1_h, c1_h, s2_h, c2_h,
             acc, cnt, sidx, didx, r0, r1, onesb, cbuf, g0, g1, q0, q1):
        cid = lax.axis_index("c")
        sid = lax.axis_index("s")
        rbufs = (r0, r1)
        gsems = (g0, g1)
        ssems = (q0, q1)
        _init_ones(onesb)
        row0 = sid * (EPT // CH)

        @pl.when(cid == 0)
        def _ii():
            _agg(si_h, di_h, x_h, s1_h, c1_h, acc, cnt, sidx, didx, rbufs,
                 onesb, cbuf, gsems, ssems, sid, row0, EPT // CH // SLAB,
                 True)

        @pl.when(cid == 1)
        def _iu():
            _agg(su_h, du_h, x_h, s2_h, c2_h, acc, cnt, sidx, didx, rbufs,
                 onesb, cbuf, gsems, ssems, sid, row0, EPT // CH // SLAB,
                 True)

    return body(si, di, su, du, x)


def _sc_one_agg(su, du, x):
    @functools.partial(
        pl.kernel,
        out_type=jax.ShapeDtypeStruct((NC, ACC_ROWS, D), _f32),
        mesh=_mesh(),
        scratch_types=[
            pltpu.VMEM_SHARED((ACC_ROWS, D), _f32),
            pltpu.VMEM((SLAB, CH), jnp.int32),
            pltpu.VMEM((SLAB, CH), jnp.int32),
            pltpu.VMEM((CH, D), _f32),
            pltpu.VMEM((CH, D), _f32),
            pltpu.SemaphoreType.DMA,
            pltpu.SemaphoreType.DMA,
            pltpu.SemaphoreType.DMA,
            pltpu.SemaphoreType.DMA,
        ],
    )
    def body(su_h, du_h, x_h, s3_h, acc, sidx, didx, r0, r1, g0, g1, q0, q1):
        cid = lax.axis_index("c")
        sid = lax.axis_index("s")
        wid = cid * NS + sid
        row0 = wid * (EPT // NC // CH)
        _agg(su_h, du_h, x_h, s3_h.at[cid], None, acc, None, sidx, didx,
             (r0, r1), None, None, (g0, g1), (q0, q1), sid, row0,
             EPT // NC // CH // SLAB, False)

    return body(su, du, x)


def _row_spec():
    return pl.BlockSpec((RB, D), lambda i: (i, 0))


def _full_spec(shape):
    nd = len(shape)
    return pl.BlockSpec(shape, lambda i: (0,) * nd)


def _cnt_spec():
    return pl.BlockSpec((1, RB), lambda i: (0, i))


_GRID = ((N + RB - 1) // RB,)


def _tc_mid(t1, c1, w1l, t2, c2, w2l, xi, xu, w1r, w2r, w3r, b1, b2, b3):
    def body(t1_r, c1_r, w1l_r, t2_r, c2_r, w2l_r, xi_r, xu_r, w1r_r,
             w2r_r, w3r_r, b1_r, b2_r, b3_r, ix_r, r3_r):
        m1 = t1_r[...] / jnp.maximum(c1_r[...][0], 1.0)[:, None]
        r1 = jnp.dot(xi_r[...], w1r_r[...],
                     preferred_element_type=_f32) + b1_r[...]
        ix_r[...] = jnp.maximum(
            jnp.dot(m1, w1l_r[...], preferred_element_type=_f32) + r1, 0.0)
        m2 = t2_r[...] / jnp.maximum(c2_r[...][0], 1.0)[:, None]
        r2 = jnp.dot(xu_r[...], w2r_r[...],
                     preferred_element_type=_f32) + b2_r[...]
        u2 = jnp.maximum(
            jnp.dot(m2, w2l_r[...], preferred_element_type=_f32) + r2, 0.0)
        r3_r[...] = jnp.dot(u2, w3r_r[...],
                            preferred_element_type=_f32) + b3_r[...]

    o = jax.ShapeDtypeStruct((N, D), _f32)
    return pl.pallas_call(
        body,
        grid=_GRID,
        in_specs=[_row_spec(), _cnt_spec(), _full_spec((D, D)),
                  _row_spec(), _cnt_spec(), _full_spec((D, D)),
                  _row_spec(), _row_spec(), _full_spec((D, D)),
                  _full_spec((D, D)), _full_spec((D, D)),
                  _full_spec((1, D)), _full_spec((1, D)), _full_spec((1, D))],
        out_specs=[_row_spec()] * 2,
        out_shape=[o, o],
    )(t1, c1, w1l, t2, c2, w2l, xi, xu, w1r, w2r, w3r, b1, b2, b3)


def _tc_final(t3p, c2, r3, w3l, wlin, blin):
    def body(t3_r, c2_r, r3_r, w3l_r, wl_r, bl_r, out_r):
        t3 = t3_r[...][0] + t3_r[...][1]
        m3 = t3 / jnp.maximum(c2_r[...][0], 1.0)[:, None]
        u3 = jnp.maximum(
            jnp.dot(m3, w3l_r[...], preferred_element_type=_f32) + r3_r[...],
            0.0)
        out_r[...] = jnp.dot(u3, wl_r[...],
                             preferred_element_type=_f32) + bl_r[...]

    return pl.pallas_call(
        body,
        grid=_GRID,
        in_specs=[pl.BlockSpec((NC, RB, D), lambda i: (0, i, 0)), _cnt_spec(),
                  _row_spec(), _full_spec((D, D)), _full_spec((D, D)),
                  _full_spec((1, D))],
        out_specs=_row_spec(),
        out_shape=jax.ShapeDtypeStruct((N, D), _f32),
    )(t3p, c2, r3, w3l, wlin, blin)


def _prep_edges(edge_index):
    src = edge_index[0].astype(jnp.int32)
    dst = edge_index[1].astype(jnp.int32)
    npad = E_PAD - src.shape[0]
    ar = jnp.arange(npad, dtype=jnp.int32)
    pad_src = ar % N
    pad_dst = N + ar % (ACC_ROWS - N)
    src = jnp.concatenate([src, pad_src]).reshape(IDXROWS, CH)
    dst = jnp.concatenate([dst, pad_dst]).reshape(IDXROWS, CH)
    return src, dst


def kernel(x_item, x_user, edge_index_ii, edge_index_iu, W1l, b1l, W1r,
           W2l, b2l, W2r, W3l, b3l, W3r, Wlin, blin):
    si, di = _prep_edges(edge_index_ii)
    su, du = _prep_edges(edge_index_iu)
    b1 = b1l.reshape(1, D)
    b2 = b2l.reshape(1, D)
    b3 = b3l.reshape(1, D)
    bl = blin.reshape(1, D)

    t1, c1, t2, c2 = _sc_two_aggs(si, di, su, du, x_item)
    c1 = c1.reshape(1, ACC_ROWS)
    c2 = c2.reshape(1, ACC_ROWS)
    item_x, r3 = _tc_mid(t1, c1, W1l.T, t2, c2, W2l.T, x_item, x_user,
                         W1r.T, W2r.T, W3r.T, b1, b2, b3)
    t3p = _sc_one_agg(su, du, item_x)
    return _tc_final(t3p, c2, r3, W3l.T, Wlin.T, bl)


# speedup vs baseline: 1.1718x; 1.0099x over previous
import functools

import jax
import jax.numpy as jnp
from jax import lax
from jax.experimental import pallas as pl
from jax.experimental.pallas import tpu as pltpu
from jax.experimental.pallas import tpu_sc as plsc

N = 10000
D = 128
NC = 2
NS = 16
NW = NC * NS
CH = 128
SLAB = 56
STAGES1 = (56, 56, 48)
STAGES3 = (56, 24)
EPT = 20480
E_PAD = NS * EPT
IDXROWS = E_PAD // CH
ACC_ROWS = 10112
RPT = ACC_ROWS // NS
RB = 2048

_f32 = jnp.float32

_SLABS = [(o, min(CH, RPT - o)) for o in range(0, RPT, CH)]


def _zero_rbuf(rb):
    @pl.loop(0, CH * (D // 16))
    def _z(i):
        rb[i // (D // 16), pl.ds((i % (D // 16)) * 16, 16)] = (
            jnp.zeros((16,), _f32))


def _agg(src_h, dst_h, tbl_h, s_out, c_out, acc, cnt, sidx, didx, rbufs,
         onesb, cbuf, gsems, ssems, sid, row0, stages, with_counts):
    _zero_rbuf(rbufs[0])
    for off, n in _SLABS:
        pltpu.sync_copy(rbufs[0].at[pl.ds(0, n), :],
                        acc.at[pl.ds(sid * RPT + off, n), :])
    if with_counts:
        for off, n in _SLABS:
            pltpu.sync_copy(rbufs[0].at[0, pl.ds(0, n)],
                            cnt.at[pl.ds(sid * RPT + off, n)])
    plsc.subcore_barrier()

    r0 = row0
    for sn in stages:
        pltpu.sync_copy(src_h.at[pl.ds(r0, sn), :], sidx.at[pl.ds(0, sn), :])
        pltpu.sync_copy(dst_h.at[pl.ds(r0, sn), :], didx.at[pl.ds(0, sn), :])
        r0 += sn
        for b in range(2):
            pltpu.async_copy(tbl_h.at[sidx.at[b]], rbufs[b], gsems[b])

        @pl.loop(0, sn // 2)
        def _pair(p):
            for b in range(2):
                j = 2 * p + b
                pltpu.make_async_copy(
                    tbl_h.at[sidx.at[b]], rbufs[b], gsems[b]).wait()
                pltpu.sync_copy(rbufs[b], acc.at[didx.at[j]], add=True)
                if with_counts:
                    pltpu.sync_copy(onesb, cnt.at[didx.at[j]], add=True)

                @pl.when(j + 2 < sn)
                def _pf():
                    pltpu.async_copy(tbl_h.at[sidx.at[j + 2]], rbufs[b],
                                     gsems[b])

    plsc.subcore_barrier()
    pltpu.sync_copy(acc.at[pl.ds(sid * RPT, RPT), :],
                    s_out.at[pl.ds(sid * RPT, RPT), :])
    if with_counts:
        pltpu.sync_copy(cnt.at[pl.ds(sid * RPT, RPT)], cbuf)
        pltpu.sync_copy(cbuf, c_out.at[pl.ds(sid * RPT, RPT)])
    plsc.subcore_barrier()


def _init_ones(onesb):
    @pl.loop(0, CH // 16)
    def _o(i):
        onesb[pl.ds(i * 16, 16)] = jnp.ones((16,), _f32)


def _mesh():
    return plsc.VectorSubcoreMesh(core_axis_name="c", subcore_axis_name="s")


def _sc_two_aggs(si, di, su, du, x):
    @functools.partial(
        pl.kernel,
        out_type=(
            jax.ShapeDtypeStruct((ACC_ROWS, D), _f32),
            jax.ShapeDtypeStruct((ACC_ROWS,), _f32),
            jax.ShapeDtypeStruct((ACC_ROWS, D), _f32),
            jax.ShapeDtypeStruct((ACC_ROWS,), _f32),
        ),
        mesh=_mesh(),
        scratch_types=[
            pltpu.VMEM_SHARED((ACC_ROWS, D), _f32),
            pltpu.VMEM_SHARED((ACC_ROWS,), _f32),
            pltpu.VMEM((SLAB, CH), jnp.int32),
            pltpu.VMEM((SLAB, CH), jnp.int32),
            pltpu.VMEM((CH, D), _f32),
            pltpu.VMEM((CH, D), _f32),
            pltpu.VMEM((CH,), _f32),
            pltpu.VMEM((RPT,), _f32),
            pltpu.SemaphoreType.DMA,
            pltpu.SemaphoreType.DMA,
            pltpu.SemaphoreType.DMA,
            pltpu.SemaphoreType.DMA,
        ],
    )
    def body(si_h, di_h, su_h, du_h, x_h, s1_h, c1_h, s2_h, c2_h,
             acc, cnt, sidx, didx, r0, r1, onesb, cbuf, g0, g1, q0, q1):
        cid = lax.axis_index("c")
        sid = lax.axis_index("s")
        rbufs = (r0, r1)
        gsems = (g0, g1)
        ssems = (q0, q1)
        _init_ones(onesb)
        row0 = sid * (EPT // CH)

        @pl.when(cid == 0)
        def _ii():
            _agg(si_h, di_h, x_h, s1_h, c1_h, acc, cnt, sidx, didx, rbufs,
                 onesb, cbuf, gsems, ssems, sid, row0, STAGES1, True)

        @pl.when(cid == 1)
        def _iu():
            _agg(su_h, du_h, x_h, s2_h, c2_h, acc, cnt, sidx, didx, rbufs,
                 onesb, cbuf, gsems, ssems, sid, row0, STAGES1, True)

    return body(si, di, su, du, x)


def _sc_one_agg(su, du, x):
    @functools.partial(
        pl.kernel,
        out_type=jax.ShapeDtypeStruct((NC, ACC_ROWS, D), _f32),
        mesh=_mesh(),
        scratch_types=[
            pltpu.VMEM_SHARED((ACC_ROWS, D), _f32),
            pltpu.VMEM((SLAB, CH), jnp.int32),
            pltpu.VMEM((SLAB, CH), jnp.int32),
            pltpu.VMEM((CH, D), _f32),
            pltpu.VMEM((CH, D), _f32),
            pltpu.SemaphoreType.DMA,
            pltpu.SemaphoreType.DMA,
            pltpu.SemaphoreType.DMA,
            pltpu.SemaphoreType.DMA,
        ],
    )
    def body(su_h, du_h, x_h, s3_h, acc, sidx, didx, r0, r1, g0, g1, q0, q1):
        cid = lax.axis_index("c")
        sid = lax.axis_index("s")
        wid = cid * NS + sid
        row0 = wid * (EPT // NC // CH)
        _agg(su_h, du_h, x_h, s3_h.at[cid], None, acc, None, sidx, didx,
             (r0, r1), None, None, (g0, g1), (q0, q1), sid, row0,
             STAGES3, False)

    return body(su, du, x)


def _row_spec():
    return pl.BlockSpec((RB, D), lambda i: (i, 0))


def _full_spec(shape):
    nd = len(shape)
    return pl.BlockSpec(shape, lambda i: (0,) * nd)


def _cnt_spec():
    return pl.BlockSpec((1, RB), lambda i: (0, i))


_GRID = ((N + RB - 1) // RB,)


def _tc_mid(t1, c1, w1l, t2, c2, w2l, xi, xu, w1r, w2r, w3r, b1, b2, b3):
    def body(t1_r, c1_r, w1l_r, t2_r, c2_r, w2l_r, xi_r, xu_r, w1r_r,
             w2r_r, w3r_r, b1_r, b2_r, b3_r, ix_r, r3_r):
        m1 = t1_r[...] / jnp.maximum(c1_r[...][0], 1.0)[:, None]
        r1 = jnp.dot(xi_r[...], w1r_r[...],
                     preferred_element_type=_f32) + b1_r[...]
        ix_r[...] = jnp.maximum(
            jnp.dot(m1, w1l_r[...], preferred_element_type=_f32) + r1, 0.0)
        m2 = t2_r[...] / jnp.maximum(c2_r[...][0], 1.0)[:, None]
        r2 = jnp.dot(xu_r[...], w2r_r[...],
                     preferred_element_type=_f32) + b2_r[...]
        u2 = jnp.maximum(
            jnp.dot(m2, w2l_r[...], preferred_element_type=_f32) + r2, 0.0)
        r3_r[...] = jnp.dot(u2, w3r_r[...],
                            preferred_element_type=_f32) + b3_r[...]

    o = jax.ShapeDtypeStruct((N, D), _f32)
    return pl.pallas_call(
        body,
        grid=_GRID,
        in_specs=[_row_spec(), _cnt_spec(), _full_spec((D, D)),
                  _row_spec(), _cnt_spec(), _full_spec((D, D)),
                  _row_spec(), _row_spec(), _full_spec((D, D)),
                  _full_spec((D, D)), _full_spec((D, D)),
                  _full_spec((1, D)), _full_spec((1, D)), _full_spec((1, D))],
        out_specs=[_row_spec()] * 2,
        out_shape=[o, o],
    )(t1, c1, w1l, t2, c2, w2l, xi, xu, w1r, w2r, w3r, b1, b2, b3)


def _tc_final(t3p, c2, r3, w3l, wlin, blin):
    def body(t3_r, c2_r, r3_r, w3l_r, wl_r, bl_r, out_r):
        t3 = t3_r[...][0] + t3_r[...][1]
        m3 = t3 / jnp.maximum(c2_r[...][0], 1.0)[:, None]
        u3 = jnp.maximum(
            jnp.dot(m3, w3l_r[...], preferred_element_type=_f32) + r3_r[...],
            0.0)
        out_r[...] = jnp.dot(u3, wl_r[...],
                             preferred_element_type=_f32) + bl_r[...]

    return pl.pallas_call(
        body,
        grid=_GRID,
        in_specs=[pl.BlockSpec((NC, RB, D), lambda i: (0, i, 0)), _cnt_spec(),
                  _row_spec(), _full_spec((D, D)), _full_spec((D, D)),
                  _full_spec((1, D))],
        out_specs=_row_spec(),
        out_shape=jax.ShapeDtypeStruct((N, D), _f32),
    )(t3p, c2, r3, w3l, wlin, blin)


def _prep_edges(edge_index):
    src = edge_index[0].astype(jnp.int32)
    dst = edge_index[1].astype(jnp.int32)
    npad = E_PAD - src.shape[0]
    ar = jnp.arange(npad, dtype=jnp.int32)
    pad_src = ar % N
    pad_dst = N + ar % (ACC_ROWS - N)
    src = jnp.concatenate([src, pad_src]).reshape(IDXROWS, CH)
    dst = jnp.concatenate([dst, pad_dst]).reshape(IDXROWS, CH)
    return src, dst


def kernel(x_item, x_user, edge_index_ii, edge_index_iu, W1l, b1l, W1r,
           W2l, b2l, W2r, W3l, b3l, W3r, Wlin, blin):
    si, di = _prep_edges(edge_index_ii)
    su, du = _prep_edges(edge_index_iu)
    b1 = b1l.reshape(1, D)
    b2 = b2l.reshape(1, D)
    b3 = b3l.reshape(1, D)
    bl = blin.reshape(1, D)

    t1, c1, t2, c2 = _sc_two_aggs(si, di, su, du, x_item)
    c1 = c1.reshape(1, ACC_ROWS)
    c2 = c2.reshape(1, ACC_ROWS)
    item_x, r3 = _tc_mid(t1, c1, W1l.T, t2, c2, W2l.T, x_item, x_user,
                         W1r.T, W2r.T, W3r.T, b1, b2, b3)
    t3p = _sc_one_agg(su, du, item_x)
    return _tc_final(t3p, c2, r3, W3l.T, Wlin.T, bl)
